# baseline (speedup 1.0000x reference)
import functools

import jax
import jax.numpy as jnp
from jax import lax
from jax.experimental import pallas as pl
from jax.experimental.pallas import tpu as pltpu
from jax.experimental.pallas import tpu_sc as plsc

DIM = 1024
INTER = 512
E = 8
SHARED_INTER = 1024
T = 2048
TB = 256
BM = 256
BMSH = 8
NCHUNK = 32
CHT = T // NCHUNK
NBLK = 23
NSLOT = NBLK * BM


def _silu(a):
    return a * jax.nn.sigmoid(a)


def _dot(a, b):
    return jax.lax.dot_general(a, b, (((1,), (1,)), ((), ())),
                               preferred_element_type=jnp.float32)


def _gating_body(s_ref, p0_ref, p1_ref, wv1_ref, wv2_ref, be_ref):
    scores = s_ref[...]
    s = scores - jnp.max(scores, axis=-1, keepdims=True)
    es = jnp.exp(s)
    probs = es / jnp.sum(es, axis=-1, keepdims=True)
    eidx = jax.lax.broadcasted_iota(jnp.int32, (T, E), 1)
    m1 = jnp.max(probs, axis=-1, keepdims=True)
    a1 = jnp.min(jnp.where(probs == m1, eidx, E), axis=-1, keepdims=True)
    sel1 = eidx == a1
    probs_m = jnp.where(sel1, -1.0, probs)
    m2 = jnp.max(probs_m, axis=-1, keepdims=True)
    a2 = jnp.min(jnp.where(probs_m == m2, eidx, E), axis=-1, keepdims=True)
    sel2 = eidx == a2
    wv1_ref[...] = m1
    wv2_ref[...] = m2
    oh = (sel1 | sel2).astype(jnp.float32)
    SB = 128
    hi = jax.lax.Precision.HIGHEST
    ltri = (jax.lax.broadcasted_iota(jnp.int32, (SB, SB), 1)
            < jax.lax.broadcasted_iota(jnp.int32, (SB, SB), 0)
            ).astype(jnp.float32)
    ones_row = jnp.ones((1, SB), jnp.float32)
    carry = jnp.zeros((1, E), jnp.float32)
    parts = []
    for k in range(T // SB):
        blk = jax.lax.slice(oh, (k * SB, 0), ((k + 1) * SB, E))
        c = jax.lax.dot_general(ltri, blk,
                                (((1,), (0,)), ((), ()))) + carry
        parts.append(c)
        carry = carry + jax.lax.dot_general(ones_row, blk,
                                            (((1,), (0,)), ((), ())))
    cx = jnp.concatenate(parts, axis=0)
    totals_i = carry.astype(jnp.int32)
    pc = (jax.lax.shift_left(
        jax.lax.shift_right_logical(totals_i + (BM - 1), BMSH), BMSH)
          ).astype(jnp.float32)
    upper = (jax.lax.broadcasted_iota(jnp.int32, (E, E), 0)
             < jax.lax.broadcasted_iota(jnp.int32, (E, E), 1)).astype(
                 jnp.float32)
    po = jax.lax.dot_general(pc, upper, (((1,), (0,)), ((), ())),
                             precision=hi)
    slot = po + cx
    p0_ref[...] = jnp.sum(jnp.where(sel1, slot, 0.0), axis=-1,
                          keepdims=True).astype(jnp.int32)
    p1_ref[...] = jnp.sum(jnp.where(sel2, slot, 0.0), axis=-1,
                          keepdims=True).astype(jnp.int32)
    ends = po + pc
    bidx = (jax.lax.broadcasted_iota(jnp.int32, (64, E), 0)
            * BM).astype(jnp.float32)
    be_ref[...] = jnp.sum((bidx >= ends).astype(jnp.int32), axis=1,
                          keepdims=True)


def _gating(scores):
    return pl.pallas_call(
        _gating_body,
        in_specs=[pl.BlockSpec((T, E), lambda: (0, 0))],
        out_specs=[pl.BlockSpec((T, 1), lambda: (0, 0)),
                   pl.BlockSpec((T, 1), lambda: (0, 0)),
                   pl.BlockSpec((T, 1), lambda: (0, 0)),
                   pl.BlockSpec((T, 1), lambda: (0, 0)),
                   pl.BlockSpec((64, 1), lambda: (0, 0))],
        out_shape=[jax.ShapeDtypeStruct((T, 1), jnp.int32),
                   jax.ShapeDtypeStruct((T, 1), jnp.int32),
                   jax.ShapeDtypeStruct((T, 1), jnp.float32),
                   jax.ShapeDtypeStruct((T, 1), jnp.float32),
                   jax.ShapeDtypeStruct((64, 1), jnp.int32)],
    )(scores)


def _splat(vec, i):
    lane = lax.iota(jnp.int32, 16)
    return jnp.max(jnp.where(lane == i, vec, jnp.full((16,), -(2 ** 30),
                                                      vec.dtype)))


def _dispatch_body(x_hbm, p0_hbm, p1_hbm, xs_hbm, xrows, p0v, p1v,
                   sem0, sem1):
    wid = lax.axis_index("s") * 2 + lax.axis_index("c")
    base = wid * CHT
    xcp = pltpu.make_async_copy(x_hbm.at[pl.ds(base, CHT)], xrows, sem0)
    xcp.start()
    pltpu.sync_copy(p0_hbm.at[wid], p0v)
    pltpu.sync_copy(p1_hbm.at[wid], p1v)
    xcp.wait()
    cp0 = pltpu.make_async_copy(xrows, xs_hbm.at[p0v], sem0)
    cp1 = pltpu.make_async_copy(xrows, xs_hbm.at[p1v], sem1)
    cp0.start()
    cp1.start()
    cp0.wait()
    cp1.wait()


def _dispatch(xt, p0, p1):
    mesh = plsc.VectorSubcoreMesh(core_axis_name="c", subcore_axis_name="s")
    f = functools.partial(
        pl.kernel, mesh=mesh,
        out_type=jax.ShapeDtypeStruct((NSLOT, DIM), jnp.float32),
        scratch_types=[pltpu.VMEM((CHT, DIM), jnp.float32),
                       pltpu.VMEM((CHT,), jnp.int32),
                       pltpu.VMEM((CHT,), jnp.int32),
                       pltpu.SemaphoreType.DMA,
                       pltpu.SemaphoreType.DMA])(_dispatch_body)
    return f(xt, p0, p1)


def _grouped_body(be_ref, xs_ref, w1_ref, w2_ref, w3_ref, o_ref):
    xb = xs_ref[...]
    a = _dot(xb, w1_ref[0])
    b = _dot(xb, w3_ref[0])
    h = _silu(a) * b
    o_ref[...] = _dot(h, w2_ref[0])


def _grouped(xs, be, w1, w2, w3):
    we = lambda i, be: (jnp.minimum(be[i], E - 1), 0, 0)
    grid_spec = pltpu.PrefetchScalarGridSpec(
        num_scalar_prefetch=1,
        grid=(NBLK,),
        in_specs=[pl.BlockSpec((BM, DIM), lambda i, be: (i, 0)),
                  pl.BlockSpec((1, INTER, DIM), we),
                  pl.BlockSpec((1, DIM, INTER), we),
                  pl.BlockSpec((1, INTER, DIM), we)],
        out_specs=pl.BlockSpec((BM, DIM), lambda i, be: (i, 0)),
    )
    return pl.pallas_call(
        _grouped_body,
        grid_spec=grid_spec,
        out_shape=jax.ShapeDtypeStruct((NSLOT, DIM), jnp.float32),
    )(be, xs, w1, w2, w3)


def _combine_body(ys_hbm, z_hbm, p0_hbm, p1_hbm, wv1_hbm, wv2_hbm, yc_hbm,
                  g0, g1, zv, idx0, idx1, wv0v, wv1v, sem0, sem1, sem2):
    wid = lax.axis_index("s") * 2 + lax.axis_index("c")
    base = wid * CHT
    lane = lax.iota(jnp.int32, 16)
    for h in range(2):
        pltpu.sync_copy(p0_hbm.at[wid, pl.ds(h * 32, 32)], idx0)
        pltpu.sync_copy(p1_hbm.at[wid, pl.ds(h * 32, 32)], idx1)
        pltpu.sync_copy(wv1_hbm.at[wid, pl.ds(h * 32, 32)], wv0v)
        pltpu.sync_copy(wv2_hbm.at[wid, pl.ds(h * 32, 32)], wv1v)
        cp0 = pltpu.make_async_copy(ys_hbm.at[idx0], g0, sem0)
        cp1 = pltpu.make_async_copy(ys_hbm.at[idx1], g1, sem1)
        cp2 = pltpu.make_async_copy(z_hbm.at[pl.ds(base + h * 32, 32)], zv,
                                    sem2)
        cp0.start()
        cp1.start()
        cp2.start()
        cp0.wait()
        cp1.wait()
        cp2.wait()
        def row_body(r, carry):
            cb = pl.multiple_of(jax.lax.shift_left(
                jax.lax.shift_right_logical(r, 4), 4), 16)
            rl = jax.lax.bitwise_and(r, 15)
            wc0 = wv0v[pl.ds(cb, 16)]
            wc1 = wv1v[pl.ds(cb, 16)]
            rsplat = jnp.full((16, 1), rl, jnp.int32)
            dnums = jax.lax.GatherDimensionNumbers(
                offset_dims=(), collapsed_slice_dims=(0,),
                start_index_map=(0,))
            w0 = jax.lax.gather(
                wc0, rsplat, dnums, (1,),
                mode=jax.lax.GatherScatterMode.PROMISE_IN_BOUNDS)
            w1 = jax.lax.gather(
                wc1, rsplat, dnums, (1,),
                mode=jax.lax.GatherScatterMode.PROMISE_IN_BOUNDS)
            for c in range(DIM // 16):
                sl = pl.ds(c * 16, 16)
                g0[r, sl] = w0 * g0[r, sl] + w1 * g1[r, sl] + zv[r, sl]
            return carry

        lax.fori_loop(0, 32, row_body, 0)
        pltpu.sync_copy(g0, yc_hbm.at[pl.ds(base + h * 32, 32)])


def _combine(ys, z, p0, p1, wv1, wv2):
    mesh = plsc.VectorSubcoreMesh(core_axis_name="c", subcore_axis_name="s")
    f = functools.partial(
        pl.kernel, mesh=mesh,
        out_type=jax.ShapeDtypeStruct((T, DIM), jnp.float32),
        scratch_types=[pltpu.VMEM((32, DIM), jnp.float32),
                       pltpu.VMEM((32, DIM), jnp.float32),
                       pltpu.VMEM((32, DIM), jnp.float32),
                       pltpu.VMEM((32,), jnp.int32),
                       pltpu.VMEM((32,), jnp.int32),
                       pltpu.VMEM((32,), jnp.float32),
                       pltpu.VMEM((32,), jnp.float32),
                       pltpu.SemaphoreType.DMA,
                       pltpu.SemaphoreType.DMA,
                       pltpu.SemaphoreType.DMA])(_combine_body)
    return f(ys, z, p0, p1, wv1, wv2)


def _shared_body(x_ref, ws1_ref, ws2_ref, ws3_ref, o_ref):
    xb = x_ref[...]
    a = _dot(xb, ws1_ref[...])
    b = _dot(xb, ws3_ref[...])
    zh = _silu(a) * b
    o_ref[...] = _dot(zh, ws2_ref[...])


def _shared(xt, ws1, ws2, ws3):
    grid = (T // TB,)
    xspec = pl.BlockSpec((TB, DIM), lambda i: (i, 0))
    full = lambda s: pl.BlockSpec(s, lambda i: (0,) * len(s))
    return pl.pallas_call(
        _shared_body,
        grid=grid,
        in_specs=[xspec, full((SHARED_INTER, DIM)),
                  full((DIM, SHARED_INTER)), full((SHARED_INTER, DIM))],
        out_specs=xspec,
        out_shape=jax.ShapeDtypeStruct((T, DIM), jnp.float32),
    )(xt, ws1, ws2, ws3)


@jax.jit
def kernel(x, gate_w, w1, w2, w3, ws1, ws2, ws3):
    shape = x.shape
    xt = x.reshape(-1, DIM)
    scores = xt @ gate_w.T
    p0, p1, wv1, wv2, be2d = _gating(scores)
    p0 = p0.reshape(NCHUNK, CHT)
    p1 = p1.reshape(NCHUNK, CHT)
    wv1 = wv1.reshape(NCHUNK, CHT)
    wv2 = wv2.reshape(NCHUNK, CHT)
    be = be2d.reshape(-1)[:NBLK]
    xs = _dispatch(xt, p0, p1)
    z = _shared(xt, ws1, ws2, ws3)
    ys = _grouped(xs, be, w1, w2, w3)
    out = _combine(ys, z, p0, p1, wv1, wv2)
    return out.reshape(shape)


# speedup vs baseline: 1.3463x; 1.3463x over previous
import jax
import jax.numpy as jnp
from jax.experimental import pallas as pl

DIM = 1024
INTER = 512
E = 8
SHARED_INTER = 1024
T = 2048
TB = 256


def _silu(a):
    return a * jax.nn.sigmoid(a)


def _dot(a, b):
    return jax.lax.dot_general(a, b, (((1,), (1,)), ((), ())),
                               preferred_element_type=jnp.float32)


def _routed_body(x_ref, s_ref, w1_ref, w2_ref, w3_ref, o_ref):
    xb = x_ref[...]
    scores = s_ref[...]
    s = scores - jnp.max(scores, axis=-1, keepdims=True)
    es = jnp.exp(s)
    probs = es / jnp.sum(es, axis=-1, keepdims=True)
    eidx = jax.lax.broadcasted_iota(jnp.int32, (TB, E), 1)
    m1 = jnp.max(probs, axis=-1, keepdims=True)
    a1 = jnp.min(jnp.where(probs == m1, eidx, E), axis=-1, keepdims=True)
    sel1 = eidx == a1
    probs_m = jnp.where(sel1, -1.0, probs)
    m2 = jnp.max(probs_m, axis=-1, keepdims=True)
    a2 = jnp.min(jnp.where(probs_m == m2, eidx, E), axis=-1, keepdims=True)
    sel2 = eidx == a2
    w = probs * (sel1 | sel2).astype(probs.dtype)

    acc = jnp.zeros((TB, DIM), jnp.float32)
    for e in range(E):
        a = _dot(xb, w1_ref[e])
        b = _dot(xb, w3_ref[e])
        h = _silu(a) * b
        acc = acc + w[:, e:e + 1] * _dot(h, w2_ref[e])
    o_ref[...] = acc


def _shared_body(x_ref, y_ref, ws1_ref, ws2_ref, ws3_ref, o_ref):
    xb = x_ref[...]
    a = _dot(xb, ws1_ref[...])
    b = _dot(xb, ws3_ref[...])
    zh = _silu(a) * b
    o_ref[...] = y_ref[...] + _dot(zh, ws2_ref[...])


@jax.jit
def kernel(x, gate_w, w1, w2, w3, ws1, ws2, ws3):
    shape = x.shape
    xt = x.reshape(-1, DIM)
    scores = xt @ gate_w.T
    grid = (T // TB,)
    xspec = pl.BlockSpec((TB, DIM), lambda i: (i, 0))
    sspec = pl.BlockSpec((TB, E), lambda i: (i, 0))
    full = lambda s: pl.BlockSpec(s, lambda i: (0,) * len(s))
    y = pl.pallas_call(
        _routed_body,
        grid=grid,
        in_specs=[xspec, sspec, full((E, INTER, DIM)), full((E, DIM, INTER)),
                  full((E, INTER, DIM))],
        out_specs=xspec,
        out_shape=jax.ShapeDtypeStruct((T, DIM), jnp.float32),
    )(xt, scores, w1, w2, w3)
    out = pl.pallas_call(
        _shared_body,
        grid=grid,
        in_specs=[xspec, xspec, full((SHARED_INTER, DIM)),
                  full((DIM, SHARED_INTER)), full((SHARED_INTER, DIM))],
        out_specs=xspec,
        out_shape=jax.ShapeDtypeStruct((T, DIM), jnp.float32),
    )(xt, y, ws1, ws2, ws3)
    return out.reshape(shape)
